# baseline (speedup 1.0000x reference)
import jax
import jax.numpy as jnp
from jax import lax
from jax.experimental import pallas as pl
from jax.experimental.pallas import tpu as pltpu
from jax.experimental.pallas import tpu_sc as plsc

_DIM = 1024
_B = 4
_L = 2048
_NW = 32
_PPW = _L // _NW
_CH = 16
_NJ = _PPW // _CH
_NSTEP = _NJ * _B
_NB = 5
_LANES = 16


def _emb_body(ids_hbm, tok_hbm, pos_hbm, out_hbm, idx_v, pos_v, rows_v,
              gsem, osem, psem, isem):
    nc = 2
    wid = lax.axis_index("s") * nc + lax.axis_index("c")
    p0 = wid * _PPW

    idx_d = [
        pltpu.async_copy(ids_hbm.at[bb, pl.ds(p0, _PPW)], idx_v.at[bb], isem)
        for bb in range(_B)
    ]
    pos_d = [None] * _NJ
    for j in range(2):
        pos_d[j] = pltpu.async_copy(
            pos_hbm.at[pl.ds(p0 + j * _CH, _CH)], pos_v.at[j % 2],
            psem.at[j % 2])

    def start_gather(s):
        j, bb = divmod(s, _B)
        rb = s % _NB
        return pltpu.async_copy(
            tok_hbm.at[idx_v.at[bb, pl.ds(j * _CH, _CH)]],
            rows_v.at[rb], gsem.at[rb])

    def start_out(s):
        j, bb = divmod(s, _B)
        rb = s % _NB
        return pltpu.async_copy(
            rows_v.at[rb],
            out_hbm.at[bb, pl.ds(p0 + j * _CH, _CH)], osem.at[rb])

    gat_d = [None] * _NSTEP
    out_d = [None] * _NSTEP
    for s in range(_NB - 1):
        if s < _B:
            idx_d[s].wait()
        gat_d[s] = start_gather(s)

    for s in range(_NSTEP):
        j, bb = divmod(s, _B)
        rb = s % _NB
        if bb == 0:
            pos_d[j].wait()
        gat_d[s].wait()

        pb = j % 2

        @plsc.parallel_loop(0, _CH)
        def _(r):
            @plsc.parallel_loop(0, _DIM, step=_LANES, unroll=8)
            def _(co):
                sl = pl.ds(co, _LANES)
                plsc.addupdate(rows_v.at[rb, r, sl], pos_v[pb, r, sl])

        if bb == _B - 1 and j + 2 < _NJ:
            pos_d[j + 2] = pltpu.async_copy(
                pos_hbm.at[pl.ds(p0 + (j + 2) * _CH, _CH)],
                pos_v.at[j % 2], psem.at[j % 2])
        out_d[s] = start_out(s)
        if s + _NB - 1 < _NSTEP:
            if s >= 1:
                out_d[s - 1].wait()
            gat_d[s + _NB - 1] = start_gather(s + _NB - 1)

    for s in range(_NSTEP - _NB, _NSTEP):
        out_d[s].wait()


@jax.jit
def _emb(token_ids, tok_table, pos_table):
    mesh = plsc.VectorSubcoreMesh(core_axis_name="c", subcore_axis_name="s")
    return pl.kernel(
        _emb_body,
        out_type=jax.ShapeDtypeStruct((_B, _L, _DIM), jnp.float32),
        mesh=mesh,
        scratch_types=[
            pltpu.VMEM((_B, _PPW), jnp.int32),
            pltpu.VMEM((2, _CH, _DIM), jnp.float32),
            pltpu.VMEM((_NB, _CH, _DIM), jnp.float32),
            pltpu.SemaphoreType.DMA((_NB,)),
            pltpu.SemaphoreType.DMA((_NB,)),
            pltpu.SemaphoreType.DMA((2,)),
            pltpu.SemaphoreType.DMA,
        ],
    )(token_ids, tok_table, pos_table)


def kernel(token_ids, tok_table, pos_table):
    return _emb(token_ids.astype(jnp.int32), tok_table, pos_table)


# speedup vs baseline: 1.0223x; 1.0223x over previous
import jax
import jax.numpy as jnp
from jax import lax
from jax.experimental import pallas as pl
from jax.experimental.pallas import tpu as pltpu
from jax.experimental.pallas import tpu_sc as plsc

_DIM = 1024
_B = 4
_L = 2048
_NW = 32
_PPW = _L // _NW
_CH = 16
_NJ = _PPW // _CH
_NSTEP = _NJ * _B
_NB = 5
_LANES = 16


def _emb_body(ids_hbm, tok_hbm, pos_hbm, out_hbm, idx_v, pos_v, rows_v,
              gsem, osem, psem, isem):
    nc = 2
    wid = lax.axis_index("s") * nc + lax.axis_index("c")
    p0 = wid * _PPW

    idx_d = [
        pltpu.async_copy(ids_hbm.at[bb, pl.ds(p0, _PPW)], idx_v.at[bb], isem)
        for bb in range(_B)
    ]
    pos_d = [None] * _NJ
    for j in range(2):
        pos_d[j] = pltpu.async_copy(
            pos_hbm.at[pl.ds(p0 + j * _CH, _CH)], pos_v.at[j % 2],
            psem.at[j % 2])

    def start_gather(s):
        j, bb = divmod(s, _B)
        rb = s % _NB
        return pltpu.async_copy(
            tok_hbm.at[idx_v.at[bb, pl.ds(j * _CH, _CH)]],
            rows_v.at[rb], gsem.at[rb])

    def start_out(s):
        j, bb = divmod(s, _B)
        rb = s % _NB
        return pltpu.async_copy(
            rows_v.at[rb],
            out_hbm.at[bb, pl.ds(p0 + j * _CH, _CH)], osem.at[rb])

    gat_d = [None] * _NSTEP
    out_d = [None] * _NSTEP
    for s in range(_NB - 1):
        if s < _B:
            idx_d[s].wait()
        gat_d[s] = start_gather(s)

    for s in range(_NSTEP):
        j, bb = divmod(s, _B)
        rb = s % _NB
        if bb == 0:
            pos_d[j].wait()
        gat_d[s].wait()

        pb = j % 2

        @plsc.parallel_loop(0, _CH)
        def _(r):
            @plsc.parallel_loop(0, _DIM, step=_LANES, unroll=4)
            def _(co):
                sl = pl.ds(co, _LANES)
                plsc.addupdate(rows_v.at[rb, r, sl], pos_v[pb, r, sl])

        if bb == _B - 1 and j + 2 < _NJ:
            pos_d[j + 2] = pltpu.async_copy(
                pos_hbm.at[pl.ds(p0 + (j + 2) * _CH, _CH)],
                pos_v.at[j % 2], psem.at[j % 2])
        out_d[s] = start_out(s)
        if s + _NB - 1 < _NSTEP:
            if s >= 1:
                out_d[s - 1].wait()
            gat_d[s + _NB - 1] = start_gather(s + _NB - 1)

    for s in range(_NSTEP - _NB, _NSTEP):
        out_d[s].wait()


@jax.jit
def _emb(token_ids, tok_table, pos_table):
    mesh = plsc.VectorSubcoreMesh(core_axis_name="c", subcore_axis_name="s")
    return pl.kernel(
        _emb_body,
        out_type=jax.ShapeDtypeStruct((_B, _L, _DIM), jnp.float32),
        mesh=mesh,
        scratch_types=[
            pltpu.VMEM((_B, _PPW), jnp.int32),
            pltpu.VMEM((2, _CH, _DIM), jnp.float32),
            pltpu.VMEM((_NB, _CH, _DIM), jnp.float32),
            pltpu.SemaphoreType.DMA((_NB,)),
            pltpu.SemaphoreType.DMA((_NB,)),
            pltpu.SemaphoreType.DMA((2,)),
            pltpu.SemaphoreType.DMA,
        ],
    )(token_ids, tok_table, pos_table)


def kernel(token_ids, tok_table, pos_table):
    return _emb(token_ids.astype(jnp.int32), tok_table, pos_table)
